# baseline (speedup 1.0000x reference)
import functools

import jax
import jax.numpy as jnp
from jax import lax
from jax.experimental import pallas as pl
from jax.experimental.pallas import tpu as pltpu
from jax.experimental.pallas import tpu_sc as plsc

EMBED_DIM = 64
CHUNK = 128


@functools.partial(jax.jit, static_argnames=("total",))
def _flat_gather(idx_flat, table, total):
    info = plsc.get_sparse_core_info()
    num_workers = info.num_cores * info.num_subcores
    per_worker = total // num_workers
    n_groups = per_worker // CHUNK
    mesh = plsc.VectorSubcoreMesh(core_axis_name="c", subcore_axis_name="s")

    @functools.partial(
        pl.kernel,
        mesh=mesh,
        compiler_params=pltpu.CompilerParams(use_tc_tiling_on_sc=False),
        out_type=jax.ShapeDtypeStruct((total, 2 * EMBED_DIM), jnp.float32),
        scratch_types=[
            pltpu.VMEM((CHUNK,), jnp.int32),
            pltpu.VMEM((CHUNK, 2 * EMBED_DIM), jnp.float32),
            pltpu.SemaphoreType.DMA,
        ],
    )
    def k(idx_hbm, table_hbm, out_hbm, idx_v, rows_v, sem):
        wid = lax.axis_index("s") * info.num_cores + lax.axis_index("c")
        base = wid * per_worker

        def body(g, carry):
            off = base + g * CHUNK
            pltpu.sync_copy(idx_hbm.at[pl.ds(off, CHUNK)], idx_v)
            for j in range(CHUNK // 16):
                idx_vec = idx_v[pl.ds(j * 16, 16)]
                pltpu.async_copy(
                    table_hbm.at[idx_vec],
                    rows_v.at[pl.ds(j * 16, 16)],
                    sem,
                )
            pltpu.make_async_copy(
                table_hbm.at[pl.ds(0, CHUNK)], rows_v, sem
            ).wait()
            pltpu.sync_copy(rows_v, out_hbm.at[pl.ds(off, CHUNK)])
            return carry

        lax.fori_loop(0, n_groups, body, 0)

    return k(idx_flat, table)


def kernel(seqTensor, table):
    batch, hist = seqTensor.shape
    total = batch * hist
    idx_flat = (seqTensor.reshape(total) >> 1).astype(jnp.int32)
    out = _flat_gather(idx_flat, table.reshape(500000, 2 * EMBED_DIM), total)
    return out[:, :EMBED_DIM].reshape(batch, hist, EMBED_DIM)


# speedup vs baseline: 1.0723x; 1.0723x over previous
import functools

import jax
import jax.numpy as jnp
from jax import lax
from jax.experimental import pallas as pl
from jax.experimental.pallas import tpu as pltpu
from jax.experimental.pallas import tpu_sc as plsc

EMBED_DIM = 64
CHUNK = 128
K = 2
GROUP = K * CHUNK
NBUF = 5


@functools.partial(jax.jit, static_argnames=("total",))
def _flat_gather(idx_flat, table, total):
    info = plsc.get_sparse_core_info()
    num_workers = info.num_cores * info.num_subcores
    per_worker = total // num_workers
    n_groups = per_worker // GROUP
    n_outer = n_groups // NBUF
    mesh = plsc.VectorSubcoreMesh(core_axis_name="c", subcore_axis_name="s")

    scratch = [pltpu.VMEM((per_worker,), jnp.int32)]
    scratch += [pltpu.VMEM((GROUP, EMBED_DIM), jnp.float32)] * NBUF
    scratch += [pltpu.SemaphoreType.DMA] * (2 * NBUF)

    @functools.partial(
        pl.kernel,
        mesh=mesh,
        compiler_params=pltpu.CompilerParams(use_tc_tiling_on_sc=False),
        out_type=jax.ShapeDtypeStruct((total, EMBED_DIM), jnp.float32),
        scratch_types=scratch,
    )
    def k(idx_hbm, table_hbm, out_hbm, idx_v, *bufs_sems):
        rows = bufs_sems[:NBUF]
        gsem = bufs_sems[NBUF:2 * NBUF]
        wsem = bufs_sems[2 * NBUF:]
        wid = lax.axis_index("s") * info.num_cores + lax.axis_index("c")
        base = wid * per_worker
        pltpu.sync_copy(idx_hbm.at[pl.ds(base, per_worker)], idx_v)

        def issue_gathers(g, x):
            for j in range(K):
                off = pl.multiple_of(g * GROUP + j * CHUNK, CHUNK)
                pltpu.async_copy(
                    table_hbm.at[idx_v.at[pl.ds(off, CHUNK)]],
                    rows[x].at[pl.ds(j * CHUNK, CHUNK)],
                    gsem[x],
                )

        def drain_gathers(x):
            pltpu.make_async_copy(
                table_hbm.at[pl.ds(0, GROUP)], rows[x], gsem[x]
            ).wait()

        def issue_write(g, x):
            woff = pl.multiple_of(base + g * GROUP, GROUP)
            pltpu.async_copy(rows[x], out_hbm.at[pl.ds(woff, GROUP)], wsem[x])

        def drain_write(x):
            pltpu.make_async_copy(
                rows[x], out_hbm.at[pl.ds(base, GROUP)], wsem[x]
            ).wait()

        for b in range(NBUF - 1):
            issue_gathers(b, b)

        def body(g, x):
            @pl.when(g + NBUF - 1 < n_groups)
            def _():
                @pl.when(g >= 1)
                def _():
                    drain_write((x - 1) % NBUF)
                issue_gathers(g + NBUF - 1, (x - 1) % NBUF)

            drain_gathers(x)
            issue_write(g, x)

        def outer_body(p, carry):
            for b in range(NBUF):
                body(p * NBUF + b, b)
            return carry

        lax.fori_loop(0, n_outer, outer_body, 0)
        for b in range(NBUF):
            drain_write(b)

    return k(idx_flat, table)


def kernel(seqTensor, table):
    batch, hist = seqTensor.shape
    total = batch * hist
    idx_flat = seqTensor.reshape(total).astype(jnp.int32)
    out = _flat_gather(idx_flat, table, total)
    return out.reshape(batch, hist, EMBED_DIM)


# speedup vs baseline: 1.0726x; 1.0003x over previous
import functools

import jax
import jax.numpy as jnp
from jax import lax
from jax.experimental import pallas as pl
from jax.experimental.pallas import tpu as pltpu
from jax.experimental.pallas import tpu_sc as plsc

EMBED_DIM = 64
HIST = 200
CHUNKS = (128, 72)
NBUF = 4


@jax.jit
def _gather(seq, table):
    batch = seq.shape[0]
    info = plsc.get_sparse_core_info()
    num_workers = info.num_cores * info.num_subcores
    per_worker = batch // num_workers
    n_outer = per_worker // NBUF
    mesh = plsc.VectorSubcoreMesh(core_axis_name="c", subcore_axis_name="s")

    scratch = [pltpu.VMEM((per_worker, HIST), jnp.int32)]
    scratch += [pltpu.VMEM((HIST, EMBED_DIM), jnp.float32)] * NBUF
    scratch += [pltpu.SemaphoreType.DMA] * (2 * NBUF)

    @functools.partial(
        pl.kernel,
        mesh=mesh,
        compiler_params=pltpu.CompilerParams(use_tc_tiling_on_sc=False),
        out_type=jax.ShapeDtypeStruct((batch, HIST, EMBED_DIM), jnp.float32),
        scratch_types=scratch,
    )
    def k(seq_hbm, table_hbm, out_hbm, idx_v, *bufs_sems):
        rows = bufs_sems[:NBUF]
        gsem = bufs_sems[NBUF:2 * NBUF]
        wsem = bufs_sems[2 * NBUF:]
        wid = lax.axis_index("s") * info.num_cores + lax.axis_index("c")
        base = wid * per_worker
        pltpu.sync_copy(seq_hbm.at[pl.ds(base, per_worker)], idx_v)

        def issue_gathers(g, x):
            off = 0
            for c in CHUNKS:
                pltpu.async_copy(
                    table_hbm.at[idx_v.at[g, pl.ds(off, c)]],
                    rows[x].at[pl.ds(off, c)],
                    gsem[x],
                )
                off += c

        def drain_gathers(x):
            pltpu.make_async_copy(
                table_hbm.at[pl.ds(0, HIST)], rows[x], gsem[x]
            ).wait()

        def issue_write(g, x):
            pltpu.async_copy(rows[x], out_hbm.at[base + g], wsem[x])

        def drain_write(x):
            pltpu.make_async_copy(
                rows[x], out_hbm.at[0], wsem[x]
            ).wait()

        for b in range(NBUF - 1):
            issue_gathers(b, b)

        def body(g, x):
            @pl.when(g + NBUF - 1 < per_worker)
            def _():
                @pl.when(g >= 1)
                def _():
                    drain_write((x - 1) % NBUF)
                issue_gathers(g + NBUF - 1, (x - 1) % NBUF)

            drain_gathers(x)
            issue_write(g, x)

        def outer_body(p, carry):
            for b in range(NBUF):
                body(p * NBUF + b, b)
            return carry

        lax.fori_loop(0, n_outer, outer_body, 0)
        for b in range(NBUF):
            drain_write(b)

    return k(seq, table)


def kernel(seqTensor, table):
    return _gather(seqTensor.astype(jnp.int32), table)
